# baseline (speedup 1.0000x reference)
import functools

import jax
import jax.numpy as jnp
from jax import lax
from jax.experimental import pallas as pl
from jax.experimental.pallas import tpu as pltpu
from jax.experimental.pallas import tpu_sc as plsc

_HALF = 128
_NSUB = 16
_EK = 80


def _sc_segsum(data2, srcp, dstp, n):
    ncp = dstp.shape[0] // (_NSUB * _EK)
    ncc = ncp - 3
    rpw = (n // _NSUB) // 8 * 8
    last_rpw = n - rpw * (_NSUB - 1)
    mesh = plsc.VectorSubcoreMesh(core_axis_name="c", subcore_axis_name="s")

    @functools.partial(
        pl.kernel,
        out_type=jax.ShapeDtypeStruct((2, n, _HALF), jnp.float32),
        mesh=mesh,
        scratch_types=(
            [pltpu.VMEM_SHARED((n + 8, _HALF), jnp.float32)]
            + [pltpu.VMEM((_EK,), jnp.int32)] * 6
            + [pltpu.VMEM((_EK, _HALF), jnp.float32)] * 3
            + [pltpu.SemaphoreType.DMA] * 12
        ),
    )
    def seg_kernel(data_hbm, src_hbm, dst_hbm, out_hbm, acc,
                   si0, si1, si2, di0, di1, di2, rw0, rw1, rw2,
                   gs0, gs1, gs2, ss0, ss1, ss2, ls0, ls1, ls2,
                   ld0, ld1, ld2):
        c = lax.axis_index("c")
        s = lax.axis_index("s")
        rr = s * rpw
        nb = ncp * _EK
        woff_s = (c * _NSUB + s) * nb
        woff_d = s * nb
        SI = (si0, si1, si2)
        DI = (di0, di1, di2)
        RW = (rw0, rw1, rw2)
        GS = (gs0, gs1, gs2)
        SS = (ss0, ss1, ss2)
        LS = (ls0, ls1, ls2)
        LD = (ld0, ld1, ld2)

        @pl.when(s < _NSUB - 1)
        def _():
            pltpu.sync_copy(data_hbm.at[pl.ds(c * n + rr, rpw)],
                            acc.at[pl.ds(rr, rpw)])

        @pl.when(s == _NSUB - 1)
        def _():
            pltpu.sync_copy(data_hbm.at[pl.ds(c * n + rr, last_rpw)],
                            acc.at[pl.ds(rr, last_rpw)])

        plsc.subcore_barrier()

        def lds(b, ch):
            return pltpu.make_async_copy(
                src_hbm.at[pl.ds(woff_s + ch * _EK, _EK)], SI[b], LS[b])

        def ldd(b, ch):
            return pltpu.make_async_copy(
                dst_hbm.at[pl.ds(woff_d + ch * _EK, _EK)], DI[b], LD[b])

        def g(b):
            return pltpu.make_async_copy(data_hbm.at[SI[b]], RW[b], GS[b])

        for b in range(3):
            lds(b, b).start()
            ldd(b, b).start()
        for b in range(3):
            lds(b, b).wait()
            g(b).start()

        @pl.loop(0, ncc // 3)
        def _(j):
            k0 = 3 * j
            scs = []
            for b in range(3):
                g(b).wait()
                lds(b, k0 + b + 3).start()
                ldd(b, k0 + b).wait()
                scs.append(pltpu.async_copy(RW[b], acc.at[DI[b]], SS[b],
                                            add=True))
            for b in range(3):
                scs[b].wait()
                ldd(b, k0 + b + 3).start()
                lds(b, k0 + b + 3).wait()
                g(b).start()

        for b in range(3):
            g(b).wait()
            ldd(b, ncc + b).wait()

        plsc.subcore_barrier()

        @pl.when(s < _NSUB - 1)
        def _():
            pltpu.sync_copy(acc.at[pl.ds(rr, rpw)],
                            out_hbm.at[c, pl.ds(rr, rpw)])

        @pl.when(s == _NSUB - 1)
        def _():
            pltpu.sync_copy(acc.at[pl.ds(rr, last_rpw)],
                            out_hbm.at[c, pl.ds(rr, last_rpw)])

    return seg_kernel(data2, srcp, dstp)


def _mlp_bn_body(hs_ref, wa_ref, ba_ref, wb_ref, bb_ref, g_ref, b_ref,
                 out_ref, sum_ref, *, n, t):
    i = pl.program_id(0)
    h = jnp.concatenate([hs_ref[0], hs_ref[1]], axis=1)
    u = jnp.maximum(jnp.dot(h, wa_ref[...],
                            preferred_element_type=jnp.float32) + ba_ref[...], 0.0)
    y = jnp.maximum(jnp.dot(u, wb_ref[...],
                            preferred_element_type=jnp.float32) + bb_ref[...], 0.0)
    out_ref[0, pl.ds(i * t, t), :] = y[:, :_HALF]
    out_ref[1, pl.ds(i * t, t), :] = y[:, _HALF:]
    s_ = jnp.sum(y, axis=0, keepdims=True)

    @pl.when(i == 0)
    def _():
        sum_ref[...] = s_

    @pl.when(i > 0)
    def _():
        sum_ref[...] += s_

    nt = n // t

    @pl.when(i == nt - 1)
    def _():
        mu = sum_ref[...] / n
        vs = jnp.zeros((1, 256), jnp.float32)
        for tt in range(nt):
            dl = out_ref[0, pl.ds(tt * t, t), :] - mu[:, :_HALF]
            dr = out_ref[1, pl.ds(tt * t, t), :] - mu[:, _HALF:]
            vs = vs + jnp.concatenate(
                [jnp.sum(dl * dl, axis=0, keepdims=True),
                 jnp.sum(dr * dr, axis=0, keepdims=True)], axis=1)
        var = vs / n
        scale = lax.rsqrt(var + 1e-5) * g_ref[...]
        shift = b_ref[...] - mu * scale
        for tt in range(nt):
            for hh in range(2):
                sl = (hh, pl.ds(tt * t, t), slice(None))
                cs = (slice(None), slice(hh * _HALF, (hh + 1) * _HALF))
                out_ref[sl] = out_ref[sl] * scale[cs] + shift[cs]


def _tc_mlp_bn(hs, wa, ba, wb, bb, g, b, n, t):
    grid = (n // t,)
    full = lambda shape: pl.BlockSpec(shape, lambda i: (0,) * len(shape))
    return pl.pallas_call(
        functools.partial(_mlp_bn_body, n=n, t=t),
        grid=grid,
        in_specs=[
            pl.BlockSpec((2, t, _HALF), lambda i: (0, i, 0)),
            full((256, 256)), full((1, 256)),
            full((256, 256)), full((1, 256)),
            full((1, 256)), full((1, 256)),
        ],
        out_specs=full((2, n, _HALF)),
        out_shape=jax.ShapeDtypeStruct((2, n, _HALF), jnp.float32),
        scratch_shapes=[pltpu.VMEM((1, 256), jnp.float32)],
    )(hs, wa, ba.reshape(1, -1), wb, bb.reshape(1, -1),
      g.reshape(1, -1), b.reshape(1, -1))


def _final_body(hs_ref, wa_ref, ba_ref, wb_ref, bb_ref, g_ref, b_ref,
                batch_ref, f1w_ref, f1b_ref, f2w_ref, f2b_ref,
                out_ref, y_ref, sum_ref, *, n, t, nseg, ncls):
    i = pl.program_id(0)
    h = jnp.concatenate([hs_ref[0], hs_ref[1]], axis=1)
    u = jnp.maximum(jnp.dot(h, wa_ref[...],
                            preferred_element_type=jnp.float32) + ba_ref[...], 0.0)
    y = jnp.maximum(jnp.dot(u, wb_ref[...],
                            preferred_element_type=jnp.float32) + bb_ref[...], 0.0)
    y_ref[pl.ds(i * t, t), :] = y
    s_ = jnp.sum(y, axis=0, keepdims=True)

    @pl.when(i == 0)
    def _():
        sum_ref[...] = s_

    @pl.when(i > 0)
    def _():
        sum_ref[...] += s_

    nt = n // t

    @pl.when(i == nt - 1)
    def _():
        mu = sum_ref[...] / n
        vs = jnp.zeros((1, 256), jnp.float32)
        for tt in range(nt):
            dd = y_ref[pl.ds(tt * t, t), :] - mu
            vs = vs + jnp.sum(dd * dd, axis=0, keepdims=True)
        var = vs / n
        scale = lax.rsqrt(var + 1e-5) * g_ref[...]
        shift = b_ref[...] - mu * scale
        pooled = jnp.zeros((nseg, 256), jnp.float32)
        for tt in range(nt):
            yt = y_ref[pl.ds(tt * t, t), :] * scale + shift
            br = batch_ref[pl.ds(tt, 1), :]
            oh = (lax.broadcasted_iota(jnp.int32, (nseg, t), 0) == br
                  ).astype(jnp.float32)
            pooled = pooled + jnp.dot(oh, yt,
                                      preferred_element_type=jnp.float32)
        p1 = jnp.maximum(jnp.dot(pooled, f1w_ref[...],
                                 preferred_element_type=jnp.float32)
                         + f1b_ref[...], 0.0)
        logits = jnp.dot(p1, f2w_ref[...],
                         preferred_element_type=jnp.float32) + f2b_ref[...]
        valid = lax.broadcasted_iota(jnp.int32, (nseg, _HALF), 1) < ncls
        neg = jnp.where(valid, logits, -1e30)
        m = jnp.max(neg, axis=1, keepdims=True)
        ex = jnp.where(valid, jnp.exp(logits - m), 0.0)
        lse = m + jnp.log(jnp.sum(ex, axis=1, keepdims=True))
        out_ref[...] = logits - lse


def _tc_final(hs, wa, ba, wb, bb, g, b, batch2d, f1w, f1b, f2wp, f2bp,
              n, t, nseg, ncls):
    grid = (n // t,)
    full = lambda shape: pl.BlockSpec(shape, lambda i: (0,) * len(shape))
    return pl.pallas_call(
        functools.partial(_final_body, n=n, t=t, nseg=nseg, ncls=ncls),
        grid=grid,
        in_specs=[
            pl.BlockSpec((2, t, _HALF), lambda i: (0, i, 0)),
            full((256, 256)), full((1, 256)),
            full((256, 256)), full((1, 256)),
            full((1, 256)), full((1, 256)),
            full(batch2d.shape),
            full((256, 256)), full((1, 256)),
            full((256, _HALF)), full((1, _HALF)),
        ],
        out_specs=full((nseg, _HALF)),
        out_shape=jax.ShapeDtypeStruct((nseg, _HALF), jnp.float32),
        scratch_shapes=[pltpu.VMEM((n, 256), jnp.float32),
                        pltpu.VMEM((1, 256), jnp.float32)],
    )(hs, wa, ba.reshape(1, -1), wb, bb.reshape(1, -1),
      g.reshape(1, -1), b.reshape(1, -1), batch2d,
      f1w, f1b.reshape(1, -1), f2wp, f2bp)


def kernel(x, edge_index, batch, W1a, b1a, W1b, b1b, W2a, b2a, W2b, b2b,
           bn1_g, bn1_b, bn2_g, bn2_b, fc1_W, fc1_b, fc2_W, fc2_b):
    n, d = x.shape
    e = edge_index.shape[1]
    nseg = 64
    ncls = fc2_W.shape[1]
    t = n // 10

    src = edge_index[0]
    dst = edge_index[1]
    x2 = jnp.concatenate([x[:, :_HALF], x[:, _HALF:]], axis=0)

    epw = e // _NSUB
    ncc = 3 * (-(-epw // (3 * _EK)))
    ncp = ncc + 3
    nb = ncp * _EK
    src2 = jnp.stack([src, src + n])
    srcp = jnp.pad(src2.reshape(2, _NSUB, epw),
                   ((0, 0), (0, 0), (0, nb - epw))).reshape(-1)
    dstp = jnp.pad(dst.reshape(_NSUB, epw), ((0, 0), (0, nb - epw)),
                   constant_values=n).reshape(-1)

    hs1 = _sc_segsum(x2, srcp, dstp, n)
    h1 = _tc_mlp_bn(hs1, W1a, b1a, W1b, b1b, bn1_g, bn1_b, n, t)
    hs2 = _sc_segsum(h1.reshape(2 * n, _HALF), srcp, dstp, n)

    batch2d = batch.reshape(n // t, t)
    f2wp = jnp.zeros((256, _HALF), jnp.float32).at[:, :ncls].set(fc2_W)
    f2bp = jnp.zeros((1, _HALF), jnp.float32).at[:, :ncls].set(fc2_b)
    out = _tc_final(hs2, W2a, b2a, W2b, b2b, bn2_g, bn2_b, batch2d,
                    fc1_W, fc1_b, f2wp, f2bp, n, t, nseg, ncls)
    return out[:, :ncls]


# speedup vs baseline: 3.4704x; 3.4704x over previous
import functools

import jax
import jax.numpy as jnp
from jax import lax
from jax.experimental import pallas as pl
from jax.experimental.pallas import tpu as pltpu
from jax.experimental.pallas import tpu_sc as plsc

_HALF = 128
_NSUB = 16
_EK = 80


def _sc_segsum(data2, src2, dst, n):
    e = dst.shape[0]
    epw = e // _NSUB
    nc = epw // _EK
    rpw = (n // _NSUB) // 8 * 8
    last_rpw = n - rpw * (_NSUB - 1)
    mesh = plsc.VectorSubcoreMesh(core_axis_name="c", subcore_axis_name="s")

    @functools.partial(
        pl.kernel,
        out_type=jax.ShapeDtypeStruct((2, n, _HALF), jnp.float32),
        mesh=mesh,
        scratch_types=[
            pltpu.VMEM_SHARED((n, _HALF), jnp.float32),
            pltpu.VMEM((_EK,), jnp.int32),
            pltpu.VMEM((_EK,), jnp.int32),
            pltpu.VMEM((_EK,), jnp.int32),
            pltpu.VMEM((_EK,), jnp.int32),
            pltpu.VMEM((_EK, _HALF), jnp.float32),
            pltpu.VMEM((_EK, _HALF), jnp.float32),
            pltpu.SemaphoreType.DMA,
            pltpu.SemaphoreType.DMA,
            pltpu.SemaphoreType.DMA,
            pltpu.SemaphoreType.DMA,
            pltpu.SemaphoreType.DMA,
            pltpu.SemaphoreType.DMA,
        ],
    )
    def seg_kernel(data_hbm, src_hbm, dst_hbm, out_hbm, acc,
                   sidx0, didx0, sidx1, didx1, rows0, rows1, gsem0, gsem1,
                   ssem0, dsem0, ssem1, dsem1):
        c = lax.axis_index("c")
        s = lax.axis_index("s")
        r0 = s * rpw

        @pl.when(s < _NSUB - 1)
        def _():
            pltpu.sync_copy(data_hbm.at[pl.ds(c * n + r0, rpw)],
                            acc.at[pl.ds(r0, rpw)])

        @pl.when(s == _NSUB - 1)
        def _():
            pltpu.sync_copy(data_hbm.at[pl.ds(c * n + r0, last_rpw)],
                            acc.at[pl.ds(r0, last_rpw)])

        plsc.subcore_barrier()

        def lds(sb, sem, ch):
            return pltpu.make_async_copy(
                src_hbm.at[pl.ds(c * e + s * epw + ch * _EK, _EK)], sb, sem)

        def ldd(db, sem, ch):
            return pltpu.make_async_copy(
                dst_hbm.at[pl.ds(s * epw + ch * _EK, _EK)], db, sem)

        def g(rb, sb, sem):
            return pltpu.make_async_copy(data_hbm.at[sb], rb, sem)

        lds(sidx0, ssem0, 0).start()
        ldd(didx0, dsem0, 0).start()
        lds(sidx0, ssem0, 0).wait()
        g(rows0, sidx0, gsem0).start()
        lds(sidx1, ssem1, 1).start()
        ldd(didx1, dsem1, 1).start()
        lds(sidx1, ssem1, 1).wait()
        g(rows1, sidx1, gsem1).start()

        @pl.loop(0, nc // 2)
        def _(j):
            ce = 2 * j
            g(rows0, sidx0, gsem0).wait()
            lds(sidx0, ssem0, ce + 2).start()
            ldd(didx0, dsem0, ce).wait()
            pltpu.sync_copy(rows0, acc.at[didx0], add=True)
            ldd(didx0, dsem0, ce + 2).start()
            lds(sidx0, ssem0, ce + 2).wait()
            g(rows0, sidx0, gsem0).start()

            g(rows1, sidx1, gsem1).wait()

            @pl.when(ce + 3 < nc)
            def _():
                lds(sidx1, ssem1, ce + 3).start()

            ldd(didx1, dsem1, ce + 1).wait()
            pltpu.sync_copy(rows1, acc.at[didx1], add=True)

            @pl.when(ce + 3 < nc)
            def _():
                ldd(didx1, dsem1, ce + 3).start()
                lds(sidx1, ssem1, ce + 3).wait()
                g(rows1, sidx1, gsem1).start()

        g(rows0, sidx0, gsem0).wait()
        ldd(didx0, dsem0, nc - 1).wait()
        pltpu.sync_copy(rows0, acc.at[didx0], add=True)

        plsc.subcore_barrier()

        @pl.when(s < _NSUB - 1)
        def _():
            pltpu.sync_copy(acc.at[pl.ds(r0, rpw)],
                            out_hbm.at[c, pl.ds(r0, rpw)])

        @pl.when(s == _NSUB - 1)
        def _():
            pltpu.sync_copy(acc.at[pl.ds(r0, last_rpw)],
                            out_hbm.at[c, pl.ds(r0, last_rpw)])

    return seg_kernel(data2, src2, dst)


def _mlp_bn_body(hs_ref, wa_ref, ba_ref, wb_ref, bb_ref, g_ref, b_ref,
                 out_ref, sum_ref, *, n, t):
    i = pl.program_id(0)
    h = jnp.concatenate([hs_ref[0], hs_ref[1]], axis=1)
    u = jnp.maximum(jnp.dot(h, wa_ref[...],
                            preferred_element_type=jnp.float32) + ba_ref[...], 0.0)
    y = jnp.maximum(jnp.dot(u, wb_ref[...],
                            preferred_element_type=jnp.float32) + bb_ref[...], 0.0)
    out_ref[0, pl.ds(i * t, t), :] = y[:, :_HALF]
    out_ref[1, pl.ds(i * t, t), :] = y[:, _HALF:]
    s_ = jnp.sum(y, axis=0, keepdims=True)

    @pl.when(i == 0)
    def _():
        sum_ref[...] = s_

    @pl.when(i > 0)
    def _():
        sum_ref[...] += s_

    nt = n // t

    @pl.when(i == nt - 1)
    def _():
        mu = sum_ref[...] / n
        vs = jnp.zeros((1, 256), jnp.float32)
        for tt in range(nt):
            dl = out_ref[0, pl.ds(tt * t, t), :] - mu[:, :_HALF]
            dr = out_ref[1, pl.ds(tt * t, t), :] - mu[:, _HALF:]
            vs = vs + jnp.concatenate(
                [jnp.sum(dl * dl, axis=0, keepdims=True),
                 jnp.sum(dr * dr, axis=0, keepdims=True)], axis=1)
        var = vs / n
        scale = lax.rsqrt(var + 1e-5) * g_ref[...]
        shift = b_ref[...] - mu * scale
        for tt in range(nt):
            for hh in range(2):
                sl = (hh, pl.ds(tt * t, t), slice(None))
                cs = (slice(None), slice(hh * _HALF, (hh + 1) * _HALF))
                out_ref[sl] = out_ref[sl] * scale[cs] + shift[cs]


def _tc_mlp_bn(hs, wa, ba, wb, bb, g, b, n, t):
    grid = (n // t,)
    full = lambda shape: pl.BlockSpec(shape, lambda i: (0,) * len(shape))
    return pl.pallas_call(
        functools.partial(_mlp_bn_body, n=n, t=t),
        grid=grid,
        in_specs=[
            pl.BlockSpec((2, t, _HALF), lambda i: (0, i, 0)),
            full((256, 256)), full((1, 256)),
            full((256, 256)), full((1, 256)),
            full((1, 256)), full((1, 256)),
        ],
        out_specs=full((2, n, _HALF)),
        out_shape=jax.ShapeDtypeStruct((2, n, _HALF), jnp.float32),
        scratch_shapes=[pltpu.VMEM((1, 256), jnp.float32)],
    )(hs, wa, ba.reshape(1, -1), wb, bb.reshape(1, -1),
      g.reshape(1, -1), b.reshape(1, -1))


def _final_body(hs_ref, wa_ref, ba_ref, wb_ref, bb_ref, g_ref, b_ref,
                batch_ref, f1w_ref, f1b_ref, f2w_ref, f2b_ref,
                out_ref, y_ref, sum_ref, *, n, t, nseg, ncls):
    i = pl.program_id(0)
    h = jnp.concatenate([hs_ref[0], hs_ref[1]], axis=1)
    u = jnp.maximum(jnp.dot(h, wa_ref[...],
                            preferred_element_type=jnp.float32) + ba_ref[...], 0.0)
    y = jnp.maximum(jnp.dot(u, wb_ref[...],
                            preferred_element_type=jnp.float32) + bb_ref[...], 0.0)
    y_ref[pl.ds(i * t, t), :] = y
    s_ = jnp.sum(y, axis=0, keepdims=True)

    @pl.when(i == 0)
    def _():
        sum_ref[...] = s_

    @pl.when(i > 0)
    def _():
        sum_ref[...] += s_

    nt = n // t

    @pl.when(i == nt - 1)
    def _():
        mu = sum_ref[...] / n
        vs = jnp.zeros((1, 256), jnp.float32)
        for tt in range(nt):
            dd = y_ref[pl.ds(tt * t, t), :] - mu
            vs = vs + jnp.sum(dd * dd, axis=0, keepdims=True)
        var = vs / n
        scale = lax.rsqrt(var + 1e-5) * g_ref[...]
        shift = b_ref[...] - mu * scale
        pooled = jnp.zeros((nseg, 256), jnp.float32)
        for tt in range(nt):
            yt = y_ref[pl.ds(tt * t, t), :] * scale + shift
            br = batch_ref[pl.ds(tt, 1), :]
            oh = (lax.broadcasted_iota(jnp.int32, (nseg, t), 0) == br
                  ).astype(jnp.float32)
            pooled = pooled + jnp.dot(oh, yt,
                                      preferred_element_type=jnp.float32)
        p1 = jnp.maximum(jnp.dot(pooled, f1w_ref[...],
                                 preferred_element_type=jnp.float32)
                         + f1b_ref[...], 0.0)
        logits = jnp.dot(p1, f2w_ref[...],
                         preferred_element_type=jnp.float32) + f2b_ref[...]
        valid = lax.broadcasted_iota(jnp.int32, (nseg, _HALF), 1) < ncls
        neg = jnp.where(valid, logits, -1e30)
        m = jnp.max(neg, axis=1, keepdims=True)
        ex = jnp.where(valid, jnp.exp(logits - m), 0.0)
        lse = m + jnp.log(jnp.sum(ex, axis=1, keepdims=True))
        out_ref[...] = logits - lse


def _tc_final(hs, wa, ba, wb, bb, g, b, batch2d, f1w, f1b, f2wp, f2bp,
              n, t, nseg, ncls):
    grid = (n // t,)
    full = lambda shape: pl.BlockSpec(shape, lambda i: (0,) * len(shape))
    return pl.pallas_call(
        functools.partial(_final_body, n=n, t=t, nseg=nseg, ncls=ncls),
        grid=grid,
        in_specs=[
            pl.BlockSpec((2, t, _HALF), lambda i: (0, i, 0)),
            full((256, 256)), full((1, 256)),
            full((256, 256)), full((1, 256)),
            full((1, 256)), full((1, 256)),
            full(batch2d.shape),
            full((256, 256)), full((1, 256)),
            full((256, _HALF)), full((1, _HALF)),
        ],
        out_specs=full((nseg, _HALF)),
        out_shape=jax.ShapeDtypeStruct((nseg, _HALF), jnp.float32),
        scratch_shapes=[pltpu.VMEM((n, 256), jnp.float32),
                        pltpu.VMEM((1, 256), jnp.float32)],
    )(hs, wa, ba.reshape(1, -1), wb, bb.reshape(1, -1),
      g.reshape(1, -1), b.reshape(1, -1), batch2d,
      f1w, f1b.reshape(1, -1), f2wp, f2bp)


def kernel(x, edge_index, batch, W1a, b1a, W1b, b1b, W2a, b2a, W2b, b2b,
           bn1_g, bn1_b, bn2_g, bn2_b, fc1_W, fc1_b, fc2_W, fc2_b):
    n, d = x.shape
    e = edge_index.shape[1]
    nseg = 64
    ncls = fc2_W.shape[1]
    t = n // 10

    src = edge_index[0]
    dst = edge_index[1]
    x2 = jnp.concatenate([x[:, :_HALF], x[:, _HALF:]], axis=0)

    src2 = jnp.concatenate([src, src + n])

    hs1 = _sc_segsum(x2, src2, dst, n)
    h1 = _tc_mlp_bn(hs1, W1a, b1a, W1b, b1b, bn1_g, bn1_b, n, t)
    hs2 = _sc_segsum(h1.reshape(2 * n, _HALF), src2, dst, n)

    batch2d = batch.reshape(n // t, t)
    f2wp = jnp.zeros((256, _HALF), jnp.float32).at[:, :ncls].set(fc2_W)
    f2bp = jnp.zeros((1, _HALF), jnp.float32).at[:, :ncls].set(fc2_b)
    out = _tc_final(hs2, W2a, b2a, W2b, b2b, bn2_g, bn2_b, batch2d,
                    fc1_W, fc1_b, f2wp, f2bp, n, t, nseg, ncls)
    return out[:, :ncls]


# speedup vs baseline: 3.4783x; 1.0023x over previous
import functools

import jax
import jax.numpy as jnp
from jax import lax
from jax.experimental import pallas as pl
from jax.experimental.pallas import tpu as pltpu
from jax.experimental.pallas import tpu_sc as plsc

_HALF = 128
_NSUB = 16
_EK = 80


def _sc_segsum(data2, src2, dst, n):
    e = dst.shape[0]
    epw = e // _NSUB
    nc = epw // _EK
    rpw = (n // _NSUB) // 8 * 8
    last_rpw = n - rpw * (_NSUB - 1)
    mesh = plsc.VectorSubcoreMesh(core_axis_name="c", subcore_axis_name="s")

    @functools.partial(
        pl.kernel,
        out_type=jax.ShapeDtypeStruct((2, n, _HALF), jnp.float32),
        mesh=mesh,
        scratch_types=[
            pltpu.VMEM_SHARED((n, _HALF), jnp.float32),
            pltpu.VMEM((_EK,), jnp.int32),
            pltpu.VMEM((_EK,), jnp.int32),
            pltpu.VMEM((_EK,), jnp.int32),
            pltpu.VMEM((_EK,), jnp.int32),
            pltpu.VMEM((_EK, _HALF), jnp.float32),
            pltpu.VMEM((_EK, _HALF), jnp.float32),
            pltpu.SemaphoreType.DMA,
            pltpu.SemaphoreType.DMA,
            pltpu.SemaphoreType.DMA,
            pltpu.SemaphoreType.DMA,
            pltpu.SemaphoreType.DMA,
            pltpu.SemaphoreType.DMA,
        ],
    )
    def seg_kernel(data_hbm, src_hbm, dst_hbm, out_hbm, acc,
                   sidx0, didx0, sidx1, didx1, rows0, rows1, gsem0, gsem1,
                   ssem0, dsem0, ssem1, dsem1):
        c = lax.axis_index("c")
        s = lax.axis_index("s")
        r0 = s * rpw

        def lds(sb, sem, ch):
            return pltpu.make_async_copy(
                src_hbm.at[pl.ds(c * e + s * epw + ch * _EK, _EK)], sb, sem)

        def ldd(db, sem, ch):
            return pltpu.make_async_copy(
                dst_hbm.at[pl.ds(s * epw + ch * _EK, _EK)], db, sem)

        def g(rb, sb, sem):
            return pltpu.make_async_copy(data_hbm.at[sb], rb, sem)

        lds(sidx0, ssem0, 0).start()
        ldd(didx0, dsem0, 0).start()
        lds(sidx0, ssem0, 0).wait()
        g(rows0, sidx0, gsem0).start()
        lds(sidx1, ssem1, 1).start()
        ldd(didx1, dsem1, 1).start()
        lds(sidx1, ssem1, 1).wait()
        g(rows1, sidx1, gsem1).start()

        @pl.when(s < _NSUB - 1)
        def _():
            pltpu.sync_copy(data_hbm.at[pl.ds(c * n + r0, rpw)],
                            acc.at[pl.ds(r0, rpw)])

        @pl.when(s == _NSUB - 1)
        def _():
            pltpu.sync_copy(data_hbm.at[pl.ds(c * n + r0, last_rpw)],
                            acc.at[pl.ds(r0, last_rpw)])

        plsc.subcore_barrier()

        @pl.loop(0, nc // 2)
        def _(j):
            ce = 2 * j
            g(rows0, sidx0, gsem0).wait()
            lds(sidx0, ssem0, ce + 2).start()
            ldd(didx0, dsem0, ce).wait()
            pltpu.sync_copy(rows0, acc.at[didx0], add=True)
            ldd(didx0, dsem0, ce + 2).start()
            lds(sidx0, ssem0, ce + 2).wait()
            g(rows0, sidx0, gsem0).start()

            g(rows1, sidx1, gsem1).wait()

            @pl.when(ce + 3 < nc)
            def _():
                lds(sidx1, ssem1, ce + 3).start()

            ldd(didx1, dsem1, ce + 1).wait()
            pltpu.sync_copy(rows1, acc.at[didx1], add=True)

            @pl.when(ce + 3 < nc)
            def _():
                ldd(didx1, dsem1, ce + 3).start()
                lds(sidx1, ssem1, ce + 3).wait()
                g(rows1, sidx1, gsem1).start()

        g(rows0, sidx0, gsem0).wait()
        ldd(didx0, dsem0, nc - 1).wait()
        pltpu.sync_copy(rows0, acc.at[didx0], add=True)

        plsc.subcore_barrier()

        @pl.when(s < _NSUB - 1)
        def _():
            pltpu.sync_copy(acc.at[pl.ds(r0, rpw)],
                            out_hbm.at[c, pl.ds(r0, rpw)])

        @pl.when(s == _NSUB - 1)
        def _():
            pltpu.sync_copy(acc.at[pl.ds(r0, last_rpw)],
                            out_hbm.at[c, pl.ds(r0, last_rpw)])

    return seg_kernel(data2, src2, dst)


def _mlp_bn_body(hs_ref, wa_ref, ba_ref, wb_ref, bb_ref, g_ref, b_ref,
                 out_ref, sum_ref, *, n, t):
    i = pl.program_id(0)
    h = jnp.concatenate([hs_ref[0], hs_ref[1]], axis=1)
    u = jnp.maximum(jnp.dot(h, wa_ref[...],
                            preferred_element_type=jnp.float32) + ba_ref[...], 0.0)
    y = jnp.maximum(jnp.dot(u, wb_ref[...],
                            preferred_element_type=jnp.float32) + bb_ref[...], 0.0)
    out_ref[0, pl.ds(i * t, t), :] = y[:, :_HALF]
    out_ref[1, pl.ds(i * t, t), :] = y[:, _HALF:]
    s_ = jnp.sum(y, axis=0, keepdims=True)

    @pl.when(i == 0)
    def _():
        sum_ref[...] = s_

    @pl.when(i > 0)
    def _():
        sum_ref[...] += s_

    nt = n // t

    @pl.when(i == nt - 1)
    def _():
        mu = sum_ref[...] / n
        vs = jnp.zeros((1, 256), jnp.float32)
        for tt in range(nt):
            dl = out_ref[0, pl.ds(tt * t, t), :] - mu[:, :_HALF]
            dr = out_ref[1, pl.ds(tt * t, t), :] - mu[:, _HALF:]
            vs = vs + jnp.concatenate(
                [jnp.sum(dl * dl, axis=0, keepdims=True),
                 jnp.sum(dr * dr, axis=0, keepdims=True)], axis=1)
        var = vs / n
        scale = lax.rsqrt(var + 1e-5) * g_ref[...]
        shift = b_ref[...] - mu * scale
        for tt in range(nt):
            for hh in range(2):
                sl = (hh, pl.ds(tt * t, t), slice(None))
                cs = (slice(None), slice(hh * _HALF, (hh + 1) * _HALF))
                out_ref[sl] = out_ref[sl] * scale[cs] + shift[cs]


def _tc_mlp_bn(hs, wa, ba, wb, bb, g, b, n, t):
    grid = (n // t,)
    full = lambda shape: pl.BlockSpec(shape, lambda i: (0,) * len(shape))
    return pl.pallas_call(
        functools.partial(_mlp_bn_body, n=n, t=t),
        grid=grid,
        in_specs=[
            pl.BlockSpec((2, t, _HALF), lambda i: (0, i, 0)),
            full((256, 256)), full((1, 256)),
            full((256, 256)), full((1, 256)),
            full((1, 256)), full((1, 256)),
        ],
        out_specs=full((2, n, _HALF)),
        out_shape=jax.ShapeDtypeStruct((2, n, _HALF), jnp.float32),
        scratch_shapes=[pltpu.VMEM((1, 256), jnp.float32)],
    )(hs, wa, ba.reshape(1, -1), wb, bb.reshape(1, -1),
      g.reshape(1, -1), b.reshape(1, -1))


def _final_body(hs_ref, wa_ref, ba_ref, wb_ref, bb_ref, g_ref, b_ref,
                batch_ref, f1w_ref, f1b_ref, f2w_ref, f2b_ref,
                out_ref, y_ref, sum_ref, *, n, t, nseg, ncls):
    i = pl.program_id(0)
    h = jnp.concatenate([hs_ref[0], hs_ref[1]], axis=1)
    u = jnp.maximum(jnp.dot(h, wa_ref[...],
                            preferred_element_type=jnp.float32) + ba_ref[...], 0.0)
    y = jnp.maximum(jnp.dot(u, wb_ref[...],
                            preferred_element_type=jnp.float32) + bb_ref[...], 0.0)
    y_ref[pl.ds(i * t, t), :] = y
    s_ = jnp.sum(y, axis=0, keepdims=True)

    @pl.when(i == 0)
    def _():
        sum_ref[...] = s_

    @pl.when(i > 0)
    def _():
        sum_ref[...] += s_

    nt = n // t

    @pl.when(i == nt - 1)
    def _():
        mu = sum_ref[...] / n
        vs = jnp.zeros((1, 256), jnp.float32)
        for tt in range(nt):
            dd = y_ref[pl.ds(tt * t, t), :] - mu
            vs = vs + jnp.sum(dd * dd, axis=0, keepdims=True)
        var = vs / n
        scale = lax.rsqrt(var + 1e-5) * g_ref[...]
        shift = b_ref[...] - mu * scale
        pooled = jnp.zeros((nseg, 256), jnp.float32)
        for tt in range(nt):
            yt = y_ref[pl.ds(tt * t, t), :] * scale + shift
            br = batch_ref[pl.ds(tt, 1), :]
            oh = (lax.broadcasted_iota(jnp.int32, (nseg, t), 0) == br
                  ).astype(jnp.float32)
            pooled = pooled + jnp.dot(oh, yt,
                                      preferred_element_type=jnp.float32)
        p1 = jnp.maximum(jnp.dot(pooled, f1w_ref[...],
                                 preferred_element_type=jnp.float32)
                         + f1b_ref[...], 0.0)
        logits = jnp.dot(p1, f2w_ref[...],
                         preferred_element_type=jnp.float32) + f2b_ref[...]
        valid = lax.broadcasted_iota(jnp.int32, (nseg, _HALF), 1) < ncls
        neg = jnp.where(valid, logits, -1e30)
        m = jnp.max(neg, axis=1, keepdims=True)
        ex = jnp.where(valid, jnp.exp(logits - m), 0.0)
        lse = m + jnp.log(jnp.sum(ex, axis=1, keepdims=True))
        out_ref[...] = logits - lse


def _tc_final(hs, wa, ba, wb, bb, g, b, batch2d, f1w, f1b, f2wp, f2bp,
              n, t, nseg, ncls):
    grid = (n // t,)
    full = lambda shape: pl.BlockSpec(shape, lambda i: (0,) * len(shape))
    return pl.pallas_call(
        functools.partial(_final_body, n=n, t=t, nseg=nseg, ncls=ncls),
        grid=grid,
        in_specs=[
            pl.BlockSpec((2, t, _HALF), lambda i: (0, i, 0)),
            full((256, 256)), full((1, 256)),
            full((256, 256)), full((1, 256)),
            full((1, 256)), full((1, 256)),
            full(batch2d.shape),
            full((256, 256)), full((1, 256)),
            full((256, _HALF)), full((1, _HALF)),
        ],
        out_specs=full((nseg, _HALF)),
        out_shape=jax.ShapeDtypeStruct((nseg, _HALF), jnp.float32),
        scratch_shapes=[pltpu.VMEM((n, 256), jnp.float32),
                        pltpu.VMEM((1, 256), jnp.float32)],
    )(hs, wa, ba.reshape(1, -1), wb, bb.reshape(1, -1),
      g.reshape(1, -1), b.reshape(1, -1), batch2d,
      f1w, f1b.reshape(1, -1), f2wp, f2bp)


def kernel(x, edge_index, batch, W1a, b1a, W1b, b1b, W2a, b2a, W2b, b2b,
           bn1_g, bn1_b, bn2_g, bn2_b, fc1_W, fc1_b, fc2_W, fc2_b):
    n, d = x.shape
    e = edge_index.shape[1]
    nseg = 64
    ncls = fc2_W.shape[1]
    t = n // 10

    src = edge_index[0]
    dst = edge_index[1]
    x2 = jnp.concatenate([x[:, :_HALF], x[:, _HALF:]], axis=0)

    src2 = jnp.concatenate([src, src + n])

    hs1 = _sc_segsum(x2, src2, dst, n)
    h1 = _tc_mlp_bn(hs1, W1a, b1a, W1b, b1b, bn1_g, bn1_b, n, t)
    hs2 = _sc_segsum(h1.reshape(2 * n, _HALF), src2, dst, n)

    batch2d = batch.reshape(n // t, t)
    f2wp = jnp.zeros((256, _HALF), jnp.float32).at[:, :ncls].set(fc2_W)
    f2bp = jnp.zeros((1, _HALF), jnp.float32).at[:, :ncls].set(fc2_b)
    out = _tc_final(hs2, W2a, b2a, W2b, b2b, bn2_g, bn2_b, batch2d,
                    fc1_W, fc1_b, f2wp, f2bp, n, t, nseg, ncls)
    return out[:, :ncls]
